# Initial kernel scaffold; baseline (speedup 1.0000x reference)
import jax
import jax.numpy as jnp
from jax.experimental import pallas as pl


def kernel(feat, logits, label, W1, b1, W2, b2, W3, b3, Wc, bc, g):
    raise NotImplementedError("write your pallas kernel here")


# speedup vs baseline: 1.3424x; 1.3424x over previous
import functools

import jax
import jax.numpy as jnp
from jax.experimental import pallas as pl
from jax.experimental.pallas import tpu as pltpu

N = 65536
D_FEAT = 128
C = 20
K = 8192
H = 256

_MLP_R = 1024


def _mlp_body(fs_ref, yi_ref, w1f_ref, w1l_ref, b1_ref, w2_ref, b2_ref,
              w3_ref, b3_ref, wc_ref, bc_ref, alpha_ref, yo_ref, ye_ref):
    x = fs_ref[...]
    yi = yi_ref[...]
    h = (jnp.dot(x, w1f_ref[...], preferred_element_type=jnp.float32)
         + jnp.dot(yi, w1l_ref[...], preferred_element_type=jnp.float32)
         + b1_ref[...])
    h = jnp.maximum(h, 0.0)
    h = jnp.maximum(jnp.dot(h, w2_ref[...], preferred_element_type=jnp.float32)
                    + b2_ref[...], 0.0)
    h = h + jnp.maximum(jnp.dot(h, w3_ref[...], preferred_element_type=jnp.float32)
                        + b3_ref[...], 0.0)
    yo = jnp.dot(h, wc_ref[...], preferred_element_type=jnp.float32) + bc_ref[...]
    alpha = alpha_ref[0, 0]
    yo_ref[...] = yo
    ye_ref[...] = alpha * yi + (1.0 - alpha) * yo


def _mlp(fs, yi, W1, b1, W2, b2, W3, b3, Wc, bc, alpha):
    w1f = W1[:D_FEAT]
    w1l = W1[D_FEAT:]
    grid = (K // _MLP_R,)
    return pl.pallas_call(
        _mlp_body,
        grid=grid,
        in_specs=[
            pl.BlockSpec((_MLP_R, D_FEAT), lambda i: (i, 0)),
            pl.BlockSpec((_MLP_R, C), lambda i: (i, 0)),
            pl.BlockSpec((D_FEAT, H), lambda i: (0, 0)),
            pl.BlockSpec((C, H), lambda i: (0, 0)),
            pl.BlockSpec((1, H), lambda i: (0, 0)),
            pl.BlockSpec((H, H), lambda i: (0, 0)),
            pl.BlockSpec((1, H), lambda i: (0, 0)),
            pl.BlockSpec((H, H), lambda i: (0, 0)),
            pl.BlockSpec((1, H), lambda i: (0, 0)),
            pl.BlockSpec((H, C), lambda i: (0, 0)),
            pl.BlockSpec((1, C), lambda i: (0, 0)),
            pl.BlockSpec(memory_space=pltpu.SMEM),
        ],
        out_specs=[
            pl.BlockSpec((_MLP_R, C), lambda i: (i, 0)),
            pl.BlockSpec((_MLP_R, C), lambda i: (i, 0)),
        ],
        out_shape=[
            jax.ShapeDtypeStruct((K, C), jnp.float32),
            jax.ShapeDtypeStruct((K, C), jnp.float32),
        ],
    )(fs, yi, w1f, w1l, b1.reshape(1, H), W2, b2.reshape(1, H), W3,
      b3.reshape(1, H), Wc, bc.reshape(1, C), alpha.reshape(1, 1))


def kernel(feat, logits, label, W1, b1, W2, b2, W3, b3, Wc, bc, g):
    p = jax.nn.softmax(logits, axis=-1)
    score = -jnp.sum(p * jnp.log(p + 1e-9), axis=-1)
    _, idx = jax.lax.top_k(score, K)
    idx = jnp.sort(idx)

    fs = jnp.take(feat, idx, axis=0)
    yi = jnp.take(logits, idx, axis=0)
    alpha = jax.nn.sigmoid(g)
    yo, ye = _mlp(fs, yi, W1, b1, W2, b2, W3, b3, Wc, bc, alpha)

    yi_full = logits
    yo_full = logits.at[idx].set(yo)
    ye_full = logits.at[idx].set(ye)
    label_mask = jnp.take(label, idx, axis=0)
    return (yi, yo, ye, yi_full, yo_full, ye_full, label, label_mask)
